# baseline (speedup 1.0000x reference)
import functools

import jax
import jax.numpy as jnp
from jax import lax
from jax.experimental import pallas as pl
from jax.experimental.pallas import tpu as pltpu
from jax.experimental.pallas import tpu_sc as plsc

B = 1024
P = 10
TAIL = 11
LOUT = P + TAIL
D = 1024

NC = 2
NS = 16
NW = NC * NS
BPW = B // NW
NBUF = 2
LANES = 16
CPR = D // LANES

_mesh = plsc.VectorSubcoreMesh(core_axis_name="c", subcore_axis_name="s")


@functools.partial(
    pl.kernel,
    out_type=jax.ShapeDtypeStruct((B, LOUT, D), jnp.float32),
    mesh=_mesh,
    scratch_types=[
        pltpu.VMEM((1, TAIL, D), jnp.float32),
        pltpu.VMEM((NBUF, P, D), jnp.float32),
        pltpu.VMEM((NBUF, LOUT, D), jnp.float32),
        pltpu.SemaphoreType.DMA,
        pltpu.SemaphoreType.DMA((NBUF,)),
        pltpu.SemaphoreType.DMA((NBUF,)),
    ],
)
def _sc_fill(vtok_hbm, tail_hbm, out_hbm,
             tstage, inslot, outslot, sem_t, sem_in, sem_out):
    wid = lax.axis_index("s") * NC + lax.axis_index("c")
    base = wid * BPW

    def fire_in(batch, s):
        return pltpu.async_copy(
            vtok_hbm.at[pl.ds(base + batch, 1)],
            inslot.at[pl.ds(s, 1)],
            sem_in.at[s])

    ins = [fire_in(s, s) for s in range(NBUF)]

    pltpu.sync_copy(tail_hbm, tstage)

    def copy_tail(s):
        def body(k, _):
            r = k // CPR
            c = (k % CPR) * LANES
            outslot[s, P + r, pl.ds(c, LANES)] = tstage[0, r, pl.ds(c, LANES)]
            return 0
        lax.fori_loop(0, TAIL * CPR, body, 0)

    for s in range(NBUF):
        copy_tail(s)

    def copy_vtok(s):
        def body(k, _):
            r = k // CPR
            c = (k % CPR) * LANES
            outslot[s, r, pl.ds(c, LANES)] = inslot[s, r, pl.ds(c, LANES)]
            return 0
        lax.fori_loop(0, P * CPR, body, 0)

    outs = [None] * NBUF
    for i in range(BPW):
        s = i % NBUF
        ins[s].wait()
        if i >= NBUF:
            outs[s].wait()
        copy_vtok(s)
        if i + NBUF < BPW:
            ins[s] = fire_in(i + NBUF, s)
        outs[s] = pltpu.async_copy(
            outslot.at[pl.ds(s, 1)],
            out_hbm.at[pl.ds(base + i, 1)],
            sem_out.at[s])
    for s in range(NBUF):
        outs[s].wait()


def kernel(categories, vtok, end, zero):
    rep = jnp.where(categories[0, 11] == 0, zero, end)
    tail = jnp.concatenate([end, jnp.broadcast_to(rep, (P, D))], axis=0)
    return _sc_fill(vtok, tail.reshape(1, TAIL, D))
